# baseline (speedup 1.0000x reference)
import functools

import jax
import jax.numpy as jnp
from jax import lax
from jax.experimental import pallas as pl
from jax.experimental.pallas import tpu as pltpu

TOKEN_BLOCK = 1024
XBUFS = 3


def _gate_outer(x_hbm, w_ref, o_hbm):
    tokens, model_dim = x_hbm.shape
    num_experts = w_ref.shape[0]

    def body(x_blk, o_blk):
        o_blk[...] = lax.dot_general(
            x_blk[...], w_ref[...],
            dimension_numbers=(((1,), (1,)), ((), ())),
            preferred_element_type=jnp.float32)

    pipeline = pltpu.emit_pipeline(
        body,
        grid=(tokens // TOKEN_BLOCK,),
        in_specs=[
            pl.BlockSpec((TOKEN_BLOCK, model_dim), lambda i: (i, 0),
                         pipeline_mode=pl.Buffered(buffer_count=XBUFS)),
        ],
        out_specs=[
            pl.BlockSpec((TOKEN_BLOCK, num_experts), lambda i: (i, 0)),
        ],
    )
    pipeline(x_hbm, o_hbm)


@jax.jit
def kernel(x, W):
    tokens, model_dim = x.shape
    num_experts = W.shape[0]
    return pl.pallas_call(
        _gate_outer,
        in_specs=[
            pl.BlockSpec(memory_space=pltpu.MemorySpace.HBM),
            pl.BlockSpec((num_experts, model_dim), lambda: (0, 0)),
        ],
        out_specs=pl.BlockSpec(memory_space=pltpu.MemorySpace.HBM),
        out_shape=jax.ShapeDtypeStruct((tokens, num_experts), jnp.float32),
    )(x, W)


# speedup vs baseline: 1.0311x; 1.0311x over previous
import functools

import jax
import jax.numpy as jnp
from jax import lax
from jax.experimental import pallas as pl
from jax.experimental.pallas import tpu as pltpu

TOKEN_BLOCK = 256
XBUFS = 8


def _gate_outer(x_hbm, w_ref, o_hbm):
    tokens, model_dim = x_hbm.shape
    num_experts = w_ref.shape[0]

    def body(x_blk, o_blk):
        o_blk[...] = lax.dot_general(
            x_blk[...], w_ref[...],
            dimension_numbers=(((1,), (1,)), ((), ())),
            preferred_element_type=jnp.float32)

    pipeline = pltpu.emit_pipeline(
        body,
        grid=(tokens // TOKEN_BLOCK,),
        in_specs=[
            pl.BlockSpec((TOKEN_BLOCK, model_dim), lambda i: (i, 0),
                         pipeline_mode=pl.Buffered(buffer_count=XBUFS)),
        ],
        out_specs=[
            pl.BlockSpec((TOKEN_BLOCK, num_experts), lambda i: (i, 0)),
        ],
    )
    pipeline(x_hbm, o_hbm)


@jax.jit
def kernel(x, W):
    tokens, model_dim = x.shape
    num_experts = W.shape[0]
    return pl.pallas_call(
        _gate_outer,
        in_specs=[
            pl.BlockSpec(memory_space=pltpu.MemorySpace.HBM),
            pl.BlockSpec((num_experts, model_dim), lambda: (0, 0)),
        ],
        out_specs=pl.BlockSpec(memory_space=pltpu.MemorySpace.HBM),
        out_shape=jax.ShapeDtypeStruct((tokens, num_experts), jnp.float32),
    )(x, W)
